# Initial kernel scaffold; baseline (speedup 1.0000x reference)
import jax
import jax.numpy as jnp
from jax.experimental import pallas as pl


def kernel(x, edge_index, edge_weight, W1, b1, W2, b2):
    raise NotImplementedError("write your pallas kernel here")


# speedup vs baseline: 7.7610x; 7.7610x over previous
import functools

import jax
import jax.numpy as jnp
from jax import lax
from jax.experimental import pallas as pl
from jax.experimental.pallas import tpu as pltpu
from jax.experimental.pallas import tpu_sc as plsc

N = 10000
E = 320000
D1 = 128
D2 = 16

NC = 2
NS = 16
NW = NC * NS
EPW = E // NW
C = 80
NCH = EPW // C
IB = 25
NB = NCH // IB
NRC = N // C


def _spmm_body(d, support_hbm, src_hbm, dst_hbm, ew_hbm, out_hbm,
               srcbuf, dstbuf, ewbuf, rows, acc, sem):
    cid = lax.axis_index("c")
    sid = lax.axis_index("s")
    wid = cid * NS + sid
    nd = d // 16

    def zloop(z, _):
        for k in range(nd):
            rows[z, pl.ds(16 * k, 16)] = jnp.zeros((16,), jnp.float32)
        return _
    lax.fori_loop(0, C, zloop, None)
    my_nrc = (NRC + NS - 1 - sid) // NS

    def zcopy(r, _):
        ch = sid + NS * r
        pltpu.sync_copy(rows, acc.at[pl.ds(C * ch, C)])
        return _
    lax.fori_loop(0, my_nrc, zcopy, None)
    plsc.subcore_barrier()

    def batch(b, _):
        pltpu.sync_copy(src_hbm.at[wid, b], srcbuf)
        pltpu.sync_copy(dst_hbm.at[wid, b], dstbuf)
        pltpu.sync_copy(ew_hbm.at[wid, b], ewbuf)

        def chunk(i, _):
            pltpu.async_copy(support_hbm.at[srcbuf.at[i]], rows, sem).wait()

            def scale(g, _):
                wvec = ewbuf[i, pl.ds(16 * g, 16)]
                for j in range(16):
                    w = wvec[j]
                    e = 16 * g + j
                    for k in range(nd):
                        rows[e, pl.ds(16 * k, 16)] = (
                            rows[e, pl.ds(16 * k, 16)] * w)
                return _
            lax.fori_loop(0, C // 16, scale, None)

            pltpu.sync_copy(rows, acc.at[dstbuf.at[i]], add=True)
            return _
        lax.fori_loop(0, IB, chunk, None)
        return _
    lax.fori_loop(0, NB, batch, None)

    plsc.subcore_barrier()

    def ocopy(r, _):
        ch = sid + NS * r
        pltpu.sync_copy(acc.at[pl.ds(C * ch, C)],
                        out_hbm.at[cid, pl.ds(C * ch, C)])
        return _
    lax.fori_loop(0, my_nrc, ocopy, None)


def _make_spmm(d):
    mesh = plsc.VectorSubcoreMesh(core_axis_name="c", subcore_axis_name="s")
    return functools.partial(
        pl.kernel,
        functools.partial(_spmm_body, d),
        out_type=jax.ShapeDtypeStruct((NC, N, d), jnp.float32),
        mesh=mesh,
        scratch_types=[
            pltpu.VMEM((IB, C), jnp.int32),
            pltpu.VMEM((IB, C), jnp.int32),
            pltpu.VMEM((IB, C), jnp.float32),
            pltpu.VMEM((C, d), jnp.float32),
            pltpu.VMEM_SHARED((N, d), jnp.float32),
            pltpu.SemaphoreType.DMA,
        ],
        compiler_params=pltpu.CompilerParams(use_tc_tiling_on_sc=False),
    )()


_spmm1 = _make_spmm(D1)
_spmm2 = _make_spmm(D2)


_BN = 1000


def _mm_body(x_ref, w_ref, o_ref):
    o_ref[...] = jnp.dot(x_ref[...], w_ref[...],
                         preferred_element_type=jnp.float32)


def _mm(x, w):
    n, k = x.shape
    m = w.shape[1]
    return pl.pallas_call(
        _mm_body,
        grid=(n // _BN,),
        in_specs=[pl.BlockSpec((_BN, k), lambda i: (i, 0)),
                  pl.BlockSpec((k, m), lambda i: (0, 0))],
        out_specs=pl.BlockSpec((_BN, m), lambda i: (i, 0)),
        out_shape=jax.ShapeDtypeStruct((n, m), jnp.float32),
    )(x, w)


def _mid_body(p_ref, b_ref, w_ref, e1_ref, s2_ref):
    h = p_ref[0] + p_ref[1] + b_ref[...]
    e1_ref[...] = h
    s2_ref[...] = jnp.dot(jnp.maximum(h, 0.0), w_ref[...],
                          preferred_element_type=jnp.float32)


def _mid(p, b1, w2):
    return pl.pallas_call(
        _mid_body,
        grid=(N // _BN,),
        in_specs=[pl.BlockSpec((NC, _BN, D1), lambda i: (0, i, 0)),
                  pl.BlockSpec((1, D1), lambda i: (0, 0)),
                  pl.BlockSpec((D1, D2), lambda i: (0, 0))],
        out_specs=[pl.BlockSpec((_BN, D1), lambda i: (i, 0)),
                   pl.BlockSpec((_BN, D2), lambda i: (i, 0))],
        out_shape=[jax.ShapeDtypeStruct((N, D1), jnp.float32),
                   jax.ShapeDtypeStruct((N, D2), jnp.float32)],
    )(p, b1.reshape(1, D1), w2)


def _fin_body(q_ref, b_ref, o_ref, e2_ref):
    h = q_ref[0] + q_ref[1] + b_ref[...]
    e2_ref[...] = h
    m = jnp.max(h, axis=1, keepdims=True)
    lse = jnp.log(jnp.sum(jnp.exp(h - m), axis=1, keepdims=True)) + m
    o_ref[...] = h - lse


def _fin(q, b2):
    return pl.pallas_call(
        _fin_body,
        grid=(N // _BN,),
        in_specs=[pl.BlockSpec((NC, _BN, D2), lambda i: (0, i, 0)),
                  pl.BlockSpec((1, D2), lambda i: (0, 0))],
        out_specs=[pl.BlockSpec((_BN, D2), lambda i: (i, 0)),
                   pl.BlockSpec((_BN, D2), lambda i: (i, 0))],
        out_shape=[jax.ShapeDtypeStruct((N, D2), jnp.float32),
                   jax.ShapeDtypeStruct((N, D2), jnp.float32)],
    )(q, b2.reshape(1, D2))


def kernel(x, edge_index, edge_weight, W1, b1, W2, b2):
    src = edge_index[1].reshape(NW, NB, IB, C)
    dst = edge_index[0].reshape(NW, NB, IB, C)
    ew = edge_weight.reshape(NW, NB, IB, C)

    support1 = _mm(x, W1)
    p = _spmm1(support1, src, dst, ew)
    embed1, support2 = _mid(p, b1, W2)
    q = _spmm2(support2, src, dst, ew)
    out, embed2 = _fin(q, b2)
    return (out, embed1, embed2)
